# baseline (speedup 1.0000x reference)
import functools

import jax
import jax.numpy as jnp
from jax import lax
from jax.experimental import pallas as pl
from jax.experimental.pallas import tpu as pltpu
from jax.experimental.pallas import tpu_sc as plsc

VOCAB = 1000000
EMBED = 64
BATCH = 16384
CTX = 20

NC = 2
NS = 16
NW = NC * NS
B_PER_W = BATCH // NW
N_CHUNK = 16
CB = B_PER_W // N_CHUNK
CROWS = CB * CTX
GPC = CROWS // 128

_mesh = plsc.VectorSubcoreMesh(core_axis_name="c", subcore_axis_name="s")


@functools.partial(
    pl.kernel,
    out_type=jax.ShapeDtypeStruct((BATCH * CTX,), jnp.float32),
    mesh=_mesh,
    compiler_params=pltpu.CompilerParams(needs_layout_passes=False,
                                         use_tc_tiling_on_sc=False),
    scratch_types=[
        pltpu.VMEM((CTX * B_PER_W // 128, 128), jnp.int32),
        pltpu.VMEM((B_PER_W // 128, 128), jnp.int32),
        pltpu.VMEM((B_PER_W, EMBED), jnp.float32),
        pltpu.VMEM((CROWS, EMBED), jnp.float32),
        pltpu.VMEM((B_PER_W * CTX,), jnp.float32),
        pltpu.SemaphoreType.DMA,
    ],
)
def _skipgram_sc(tw_hbm, cw_hbm, tt_hbm, ct_hbm, out_hbm,
                 cidx, tidx, tgt_buf, ctx_buf, out_buf, sem):
    wid = lax.axis_index("s") * NC + lax.axis_index("c")

    pltpu.sync_copy(tw_hbm.at[wid], tidx)
    pltpu.sync_copy(cw_hbm.at[wid], cidx)

    for j in range(B_PER_W // 128):
        pltpu.async_copy(tt_hbm.at[tidx.at[j]],
                         tgt_buf.at[pl.ds(j * 128, 128)], sem)
    for j in range(B_PER_W // 128):
        pltpu.make_async_copy(tt_hbm.at[tidx.at[j]],
                              tgt_buf.at[pl.ds(j * 128, 128)], sem).wait()

    def chunk_body(chunk, carry):
        for j in range(GPC):
            pltpu.async_copy(ct_hbm.at[cidx.at[chunk * GPC + j]],
                             ctx_buf.at[pl.ds(j * 128, 128)], sem)
        for j in range(GPC):
            pltpu.make_async_copy(ct_hbm.at[cidx.at[chunk * GPC + j]],
                                  ctx_buf.at[pl.ds(j * 128, 128)], sem).wait()

        lane_iota = lax.iota(jnp.int32, 16)

        def group_body(g, carry2):
            accs = [jnp.zeros((16,), jnp.float32) for _ in range(5)]
            for bl in range(4):
                b = chunk * CB + g * 4 + bl
                t = [tgt_buf[b, pl.ds(k * 16, 16)]
                     for k in range(EMBED // 16)]
                for c in range(CTX):
                    r = g * (4 * CTX) + bl * CTX + c
                    p = t[0] * ctx_buf[r, pl.ds(0, 16)]
                    for k in range(1, EMBED // 16):
                        p = p + t[k] * ctx_buf[r, pl.ds(k * 16, 16)]
                    s = jnp.sum(p)
                    v, lane = divmod(bl * CTX + c, 16)
                    accs[v] = jnp.where(lane_iota == lane, s, accs[v])
            base = chunk * (CB * CTX) + g * 80
            for v in range(5):
                out_buf[pl.ds(base + v * 16, 16)] = accs[v]
            return carry2

        lax.fori_loop(0, CB // 4, group_body, 0, unroll=False)
        return carry

    lax.fori_loop(0, N_CHUNK, chunk_body, 0, unroll=False)

    pltpu.sync_copy(out_buf, out_hbm.at[pl.ds(wid * B_PER_W * CTX,
                                              B_PER_W * CTX)])


def kernel(target_words, context_words, target_table, context_table):
    tw = target_words.reshape(NW, B_PER_W // 128, 128)
    cw = context_words.reshape(NW, CTX * B_PER_W // 128, 128)
    out = _skipgram_sc(tw, cw, target_table, context_table)
    return out.reshape(BATCH, CTX)
